# Initial kernel scaffold; baseline (speedup 1.0000x reference)
import jax
import jax.numpy as jnp
from jax.experimental import pallas as pl


def kernel(mode_features, r_w1, r_b1, r_w2, r_b2, r_w3, r_b3, t_w1, t_b1, t_w2, t_b2, t_w3, t_b3, s_w1, s_b1, s_w2, s_b2, s_w3, s_b3):
    raise NotImplementedError("write your pallas kernel here")


# speedup vs baseline: 3.9775x; 3.9775x over previous
import functools

import jax
import jax.numpy as jnp
from jax.experimental import pallas as pl

B, M, D = 4096, 6, 128
E, TOPK, FUT = 6, 2, 60
N = B * M
OUT = FUT * 2


def _gelu(v):
    return v * 0.5 * (1.0 + jax.lax.erf(v * 0.7071067811865476))


def _fused_body(x_ref, r_w1, r_b1, r_w2, r_b2, r_w3, r_b3,
                t_w1, t_b1, t_w2, t_b2, t_w3, t_b3,
                s_w1, s_b1, s_w2, s_b2, s_w3, s_b3,
                traj_ref, score_ref, probs_ref, psum_ref):
    x = x_ref[...]
    f32 = jnp.float32
    dot = functools.partial(jnp.dot, preferred_element_type=f32)

    h = _gelu(dot(x, r_w1[...]) + r_b1[...])
    h = _gelu(dot(h, r_w2[...]) + r_b2[...])
    logits = dot(h, r_w3[...]) + r_b3[...]

    m = jnp.max(logits, axis=-1, keepdims=True)
    ex = jnp.exp(logits - m)
    denom = jnp.sum(ex, axis=-1, keepdims=True)
    probs = ex / denom
    probs_ref[...] = probs
    psum_ref[...] = jnp.sum(probs, axis=0).reshape(1, 1, E)

    T = x.shape[0]
    col = jax.lax.broadcasted_iota(jnp.int32, (T, E), 1)
    m0 = jnp.max(logits, axis=-1)
    i0 = jnp.min(jnp.where(logits == m0[:, None], col, E), axis=-1)
    masked = jnp.where(col == i0[:, None], -jnp.inf, logits)
    m1 = jnp.max(masked, axis=-1)
    i1 = jnp.min(jnp.where(masked == m1[:, None], col, E), axis=-1)
    e1 = jnp.exp(m1 - m0)
    p0 = 1.0 / (1.0 + e1)
    p1 = e1 / (1.0 + e1)

    acc_t = jnp.zeros((T, OUT), f32)
    acc_s = jnp.zeros((T,), f32)
    for e in range(E):
        th = _gelu(dot(x, t_w1[e]) + t_b1[e])
        th = _gelu(dot(th, t_w2[e]) + t_b2[e])
        tr = dot(th, t_w3[e]) + t_b3[e]
        sh = _gelu(dot(x, s_w1[e]) + s_b1[e])
        sh = _gelu(dot(sh, s_w2[e]) + s_b2[e])
        sc = jnp.sum(sh * s_w3[e, :, 0], axis=-1) + s_b3[e, 0]
        w = jnp.where(i0 == e, p0, 0.0) + jnp.where(i1 == e, p1, 0.0)
        acc_t = acc_t + w[:, None] * tr
        acc_s = acc_s + w * sc
    traj_ref[...] = acc_t
    score_ref[...] = acc_s[:, None]


def kernel(mode_features, r_w1, r_b1, r_w2, r_b2, r_w3, r_b3,
           t_w1, t_b1, t_w2, t_b2, t_w3, t_b3,
           s_w1, s_b1, s_w2, s_b2, s_w3, s_b3):
    x = mode_features.reshape(N, D)
    r_b1 = r_b1.reshape(1, -1)
    r_b2 = r_b2.reshape(1, -1)
    r_b3 = r_b3.reshape(1, -1)

    TILE = 1024
    grid = (N // TILE,)

    def tok_map(i):
        return (i, 0)

    def const_map2(i):
        return (0, 0)

    def const_map3(i):
        return (0, 0, 0)

    full2 = lambda a: pl.BlockSpec(a.shape, const_map2)
    full3 = lambda a: pl.BlockSpec(a.shape, const_map3)

    traj, score, probs, psum = pl.pallas_call(
        _fused_body,
        grid=grid,
        in_specs=[
            pl.BlockSpec((TILE, D), tok_map),
            full2(r_w1), full2(r_b1), full2(r_w2), full2(r_b2),
            full2(r_w3), full2(r_b3),
            full3(t_w1), full2(t_b1), full3(t_w2), full2(t_b2),
            full3(t_w3), full2(t_b3),
            full3(s_w1), full2(s_b1), full3(s_w2), full2(s_b2),
            full3(s_w3), full2(s_b3),
        ],
        out_specs=[
            pl.BlockSpec((TILE, OUT), tok_map),
            pl.BlockSpec((TILE, 1), tok_map),
            pl.BlockSpec((TILE, E), tok_map),
            pl.BlockSpec((1, 1, E), lambda i: (i, 0, 0)),
        ],
        out_shape=[
            jax.ShapeDtypeStruct((N, OUT), jnp.float32),
            jax.ShapeDtypeStruct((N, 1), jnp.float32),
            jax.ShapeDtypeStruct((N, E), jnp.float32),
            jax.ShapeDtypeStruct((grid[0], 1, E), jnp.float32),
        ],
    )(x, r_w1, r_b1, r_w2, r_b2, r_w3, r_b3,
      t_w1, t_b1, t_w2, t_b2, t_w3, t_b3,
      s_w1, s_b1, s_w2, s_b2, s_w3, s_b3)

    trajectories = traj.reshape(B, M, FUT, 2)
    scores = score.reshape(B, M)
    probs_out = probs.reshape(B, M, E)
    avg = psum.reshape(-1, E).sum(axis=0) / N
    entropy = -(avg * jnp.log(avg + 1e-08)).sum()
    load_balance_loss = -entropy * 0.01
    uniform = jnp.ones_like(avg) / E
    l2_loss = jnp.mean((avg - uniform) ** 2)
    aux_loss = load_balance_loss + 0.01 * l2_loss
    return (trajectories, scores, aux_loss, probs_out)
